# baseline (speedup 1.0000x reference)
import functools

import jax
import jax.numpy as jnp
from jax.experimental import pallas as pl

EPS = 1e-5


def _fps(xyz, npoint):
    B, N, _ = xyz.shape
    def body(i, state):
        idxs, dists, farthest = state
        idxs = idxs.at[:, i].set(farthest)
        centroid = jnp.take_along_axis(xyz, farthest[:, None, None], axis=1)
        d = jnp.sum((xyz - centroid) ** 2, axis=-1)
        dists = jnp.minimum(dists, d)
        farthest = jnp.argmax(dists, axis=-1).astype(jnp.int32)
        return (idxs, dists, farthest)
    init = (jnp.zeros((B, npoint), dtype=jnp.int32),
            jnp.full((B, N), 1e10, dtype=xyz.dtype),
            jnp.zeros((B,), dtype=jnp.int32))
    idxs, _, _ = jax.lax.fori_loop(0, npoint, body, init)
    return idxs


def _ball_query(radius, nsample, xyz, new_xyz):
    N = xyz.shape[1]
    d2 = jnp.sum((new_xyz[:, :, None, :] - xyz[:, None, :, :]) ** 2, axis=-1)
    key = jnp.where(d2 < radius * radius,
                    jnp.arange(N, dtype=jnp.int32)[None, None, :], N)
    negvals, _ = jax.lax.top_k(-key, nsample)
    vals = -negvals
    first = vals[..., :1]
    return jnp.where(vals == N, first, vals).astype(jnp.int32)


def _mlp_body(nlayers, x_ref, betas_ref, *rest):
    params = rest[:nlayers * 6]
    Wf, bf, Wh, bh = rest[nlayers * 6:nlayers * 6 + 4]
    out_ref = rest[-1]

    blk = x_ref.shape[1]
    K = x_ref.shape[2]
    C = x_ref.shape[3]
    h = x_ref[0].reshape(blk * K, C)
    for li in range(nlayers):
        W, b, g, be, m, v = (params[6 * li + j][...] for j in range(6))
        h = jax.lax.dot_general(h, W, (((1,), (1,)), ((), ())),
                                preferred_element_type=jnp.float32)
        h = h + b
        h = (h - m) / jnp.sqrt(v + EPS) * g + be
        h = jax.nn.relu(h)
    cout = h.shape[-1]
    pooled = jnp.max(h.reshape(blk, K, cout), axis=1)
    bvec = betas_ref[0]
    gamma = jax.lax.dot_general(bvec, Wf[...], (((1,), (1,)), ((), ())),
                                preferred_element_type=jnp.float32) + bf[...]
    beta = jax.lax.dot_general(bvec, Wh[...], (((1,), (1,)), ((), ())),
                               preferred_element_type=jnp.float32) + bh[...]
    out_ref[0] = pooled * gamma + beta


def _mlp_pool_film(x, betas, mlp_params, film_p, rows_blk):
    B, S, K, C = x.shape
    nlayers = len(mlp_params)
    cout = mlp_params[-1][0].shape[0]
    Wf, bf, Wh, bh = film_p

    flat_params = []
    specs = []
    for (W, b, g, be, m, v) in mlp_params:
        co = W.shape[0]
        flat_params.append(W)
        specs.append(pl.BlockSpec(W.shape, lambda b_, r_: (0, 0)))
        for p in (b, g, be, m, v):
            flat_params.append(p.reshape(1, co))
            specs.append(pl.BlockSpec((1, co), lambda b_, r_: (0, 0)))
    film_flat = [Wf, bf.reshape(1, cout), Wh, bh.reshape(1, cout)]
    film_specs = [
        pl.BlockSpec(Wf.shape, lambda b_, r_: (0, 0)),
        pl.BlockSpec((1, cout), lambda b_, r_: (0, 0)),
        pl.BlockSpec(Wh.shape, lambda b_, r_: (0, 0)),
        pl.BlockSpec((1, cout), lambda b_, r_: (0, 0)),
    ]

    betas3 = betas.reshape(B, 1, 16)
    grid = (B, S // rows_blk)
    return pl.pallas_call(
        functools.partial(_mlp_body, nlayers),
        grid=grid,
        in_specs=[
            pl.BlockSpec((1, rows_blk, K, C), lambda b_, r_: (b_, r_, 0, 0)),
            pl.BlockSpec((1, 1, 16), lambda b_, r_: (b_, 0, 0)),
        ] + specs + film_specs,
        out_specs=pl.BlockSpec((1, rows_blk, cout), lambda b_, r_: (b_, r_, 0)),
        out_shape=jax.ShapeDtypeStruct((B, S, cout), jnp.float32),
    )(x, betas3, *flat_params, *film_flat)


def _global_body(nlayers, x_ref, betas_ref, *rest):
    params = rest[:nlayers * 6]
    Wf, bf, Wh, bh, fcw, fcb = rest[nlayers * 6:nlayers * 6 + 6]
    out_ref = rest[-1]

    K = x_ref.shape[1]
    C = x_ref.shape[2]
    h = x_ref[0]
    for li in range(nlayers):
        W, b, g, be, m, v = (params[6 * li + j][...] for j in range(6))
        h = jax.lax.dot_general(h, W, (((1,), (1,)), ((), ())),
                                preferred_element_type=jnp.float32)
        h = h + b
        h = (h - m) / jnp.sqrt(v + EPS) * g + be
        h = jax.nn.relu(h)
    pooled = jnp.max(h, axis=0, keepdims=True)
    bvec = betas_ref[0]
    gamma = jax.lax.dot_general(bvec, Wf[...], (((1,), (1,)), ((), ())),
                                preferred_element_type=jnp.float32) + bf[...]
    beta = jax.lax.dot_general(bvec, Wh[...], (((1,), (1,)), ((), ())),
                               preferred_element_type=jnp.float32) + bh[...]
    f = pooled * gamma + beta
    out = jax.lax.dot_general(f, fcw[...], (((1,), (1,)), ((), ())),
                              preferred_element_type=jnp.float32) + fcb[...]
    out_ref[0] = out


def _global_stage(x, betas, mlp_params, film_p, fc_w, fc_b):
    B, K, C = x.shape
    nlayers = len(mlp_params)
    cout = mlp_params[-1][0].shape[0]
    nout = fc_w.shape[0]
    Wf, bf, Wh, bh = film_p

    flat_params = []
    specs = []
    for (W, b, g, be, m, v) in mlp_params:
        co = W.shape[0]
        flat_params.append(W)
        specs.append(pl.BlockSpec(W.shape, lambda b_: (0, 0)))
        for p in (b, g, be, m, v):
            flat_params.append(p.reshape(1, co))
            specs.append(pl.BlockSpec((1, co), lambda b_: (0, 0)))
    tail_flat = [Wf, bf.reshape(1, cout), Wh, bh.reshape(1, cout),
                 fc_w, fc_b.reshape(1, nout)]
    tail_specs = [
        pl.BlockSpec(Wf.shape, lambda b_: (0, 0)),
        pl.BlockSpec((1, cout), lambda b_: (0, 0)),
        pl.BlockSpec(Wh.shape, lambda b_: (0, 0)),
        pl.BlockSpec((1, cout), lambda b_: (0, 0)),
        pl.BlockSpec(fc_w.shape, lambda b_: (0, 0)),
        pl.BlockSpec((1, nout), lambda b_: (0, 0)),
    ]

    betas3 = betas.reshape(B, 1, 16)
    out = pl.pallas_call(
        functools.partial(_global_body, nlayers),
        grid=(B,),
        in_specs=[
            pl.BlockSpec((1, K, C), lambda b_: (b_, 0, 0)),
            pl.BlockSpec((1, 1, 16), lambda b_: (b_, 0, 0)),
        ] + specs + tail_specs,
        out_specs=pl.BlockSpec((1, 1, nout), lambda b_: (b_, 0, 0)),
        out_shape=jax.ShapeDtypeStruct((B, 1, nout), jnp.float32),
    )(x, betas3, *flat_params, *tail_flat)
    return out.reshape(B, nout)


def kernel(pointcloud, betas, sa1, sa2, sa3, film_params, fc_w, fc_b):
    xyz = pointcloud

    fidx = _fps(xyz, 512)
    new_xyz = jnp.take_along_axis(xyz, fidx[..., None].astype(jnp.int32), axis=1)
    idx = _ball_query(0.2, 64, xyz, new_xyz)
    grouped_xyz = jnp.take_along_axis(xyz[:, None, :, :], idx[..., None], axis=2) - new_xyz[:, :, None, :]
    grouped_abs = jnp.take_along_axis(xyz[:, None, :, :], idx[..., None], axis=2)
    x = jnp.concatenate([grouped_xyz, grouped_abs], axis=-1)
    f1 = _mlp_pool_film(x, betas, sa1, film_params[0], rows_blk=64)

    xyz1 = new_xyz
    fidx2 = _fps(xyz1, 256)
    new_xyz2 = jnp.take_along_axis(xyz1, fidx2[..., None].astype(jnp.int32), axis=1)
    idx2 = _ball_query(0.4, 64, xyz1, new_xyz2)
    g_xyz2 = jnp.take_along_axis(xyz1[:, None, :, :], idx2[..., None], axis=2) - new_xyz2[:, :, None, :]
    g_f2 = jnp.take_along_axis(f1[:, None, :, :], idx2[..., None], axis=2)
    x2 = jnp.concatenate([g_xyz2, g_f2], axis=-1)
    f2 = _mlp_pool_film(x2, betas, sa2, film_params[1], rows_blk=64)

    x3 = jnp.concatenate([new_xyz2, f2], axis=-1)
    return _global_stage(x3, betas, sa3, film_params[2], fc_w, fc_b)


# speedup vs baseline: 1.3405x; 1.3405x over previous
import functools

import jax
import jax.numpy as jnp
from jax import lax
from jax.experimental import pallas as pl

EPS = 1e-5


def _fps(xyz, npoint):
    B, N, _ = xyz.shape
    def body(i, state):
        idxs, dists, farthest = state
        idxs = idxs.at[:, i].set(farthest)
        centroid = jnp.take_along_axis(xyz, farthest[:, None, None], axis=1)
        d = jnp.sum((xyz - centroid) ** 2, axis=-1)
        dists = jnp.minimum(dists, d)
        farthest = jnp.argmax(dists, axis=-1).astype(jnp.int32)
        return (idxs, dists, farthest)
    init = (jnp.zeros((B, npoint), dtype=jnp.int32),
            jnp.full((B, N), 1e10, dtype=xyz.dtype),
            jnp.zeros((B,), dtype=jnp.int32))
    idxs, _, _ = jax.lax.fori_loop(0, npoint, body, init)
    return idxs


def _ball_query(radius, nsample, xyz, new_xyz):
    N = xyz.shape[1]
    d2 = jnp.sum((new_xyz[:, :, None, :] - xyz[:, None, :, :]) ** 2, axis=-1)
    key = jnp.where(d2 < radius * radius,
                    jnp.arange(N, dtype=jnp.int32)[None, None, :], N)
    negvals, _ = jax.lax.top_k(-key, nsample)
    vals = -negvals
    first = vals[..., :1]
    return jnp.where(vals == N, first, vals).astype(jnp.int32)


def _mlp_body(nlayers, x_ref, betas_ref, *rest):
    params = rest[:nlayers * 6]
    Wf, bf, Wh, bh = rest[nlayers * 6:nlayers * 6 + 4]
    out_ref = rest[-1]

    blk = x_ref.shape[1]
    K = x_ref.shape[2]
    C = x_ref.shape[3]
    h = x_ref[0].reshape(blk * K, C)
    for li in range(nlayers):
        W, b, g, be, m, v = (params[6 * li + j][...] for j in range(6))
        h = jax.lax.dot_general(h, W, (((1,), (1,)), ((), ())),
                                preferred_element_type=jnp.float32)
        h = h + b
        h = (h - m) / jnp.sqrt(v + EPS) * g + be
        h = jax.nn.relu(h)
    cout = h.shape[-1]
    pooled = jnp.max(h.reshape(blk, K, cout), axis=1)
    bvec = betas_ref[0]
    gamma = jax.lax.dot_general(bvec, Wf[...], (((1,), (1,)), ((), ())),
                                preferred_element_type=jnp.float32) + bf[...]
    beta = jax.lax.dot_general(bvec, Wh[...], (((1,), (1,)), ((), ())),
                               preferred_element_type=jnp.float32) + bh[...]
    out_ref[0] = pooled * gamma + beta


def _mlp_pool_film(x, betas, mlp_params, film_p, rows_blk):
    B, S, K, C = x.shape
    nlayers = len(mlp_params)
    cout = mlp_params[-1][0].shape[0]
    Wf, bf, Wh, bh = film_p

    flat_params = []
    specs = []
    for (W, b, g, be, m, v) in mlp_params:
        co = W.shape[0]
        flat_params.append(W)
        specs.append(pl.BlockSpec(W.shape, lambda b_, r_: (0, 0)))
        for p in (b, g, be, m, v):
            flat_params.append(p.reshape(1, co))
            specs.append(pl.BlockSpec((1, co), lambda b_, r_: (0, 0)))
    film_flat = [Wf, bf.reshape(1, cout), Wh, bh.reshape(1, cout)]
    film_specs = [
        pl.BlockSpec(Wf.shape, lambda b_, r_: (0, 0)),
        pl.BlockSpec((1, cout), lambda b_, r_: (0, 0)),
        pl.BlockSpec(Wh.shape, lambda b_, r_: (0, 0)),
        pl.BlockSpec((1, cout), lambda b_, r_: (0, 0)),
    ]

    betas3 = betas.reshape(B, 1, 16)
    grid = (B, S // rows_blk)
    return pl.pallas_call(
        functools.partial(_mlp_body, nlayers),
        grid=grid,
        in_specs=[
            pl.BlockSpec((1, rows_blk, K, C), lambda b_, r_: (b_, r_, 0, 0)),
            pl.BlockSpec((1, 1, 16), lambda b_, r_: (b_, 0, 0)),
        ] + specs + film_specs,
        out_specs=pl.BlockSpec((1, rows_blk, cout), lambda b_, r_: (b_, r_, 0)),
        out_shape=jax.ShapeDtypeStruct((B, S, cout), jnp.float32),
    )(x, betas3, *flat_params, *film_flat)


def _global_body(nlayers, x_ref, betas_ref, *rest):
    params = rest[:nlayers * 6]
    Wf, bf, Wh, bh, fcw, fcb = rest[nlayers * 6:nlayers * 6 + 6]
    out_ref = rest[-1]

    K = x_ref.shape[1]
    C = x_ref.shape[2]
    h = x_ref[0]
    for li in range(nlayers):
        W, b, g, be, m, v = (params[6 * li + j][...] for j in range(6))
        h = jax.lax.dot_general(h, W, (((1,), (1,)), ((), ())),
                                preferred_element_type=jnp.float32)
        h = h + b
        h = (h - m) / jnp.sqrt(v + EPS) * g + be
        h = jax.nn.relu(h)
    pooled = jnp.max(h, axis=0, keepdims=True)
    bvec = betas_ref[0]
    gamma = jax.lax.dot_general(bvec, Wf[...], (((1,), (1,)), ((), ())),
                                preferred_element_type=jnp.float32) + bf[...]
    beta = jax.lax.dot_general(bvec, Wh[...], (((1,), (1,)), ((), ())),
                               preferred_element_type=jnp.float32) + bh[...]
    f = pooled * gamma + beta
    out = jax.lax.dot_general(f, fcw[...], (((1,), (1,)), ((), ())),
                              preferred_element_type=jnp.float32) + fcb[...]
    out_ref[0] = out


def _global_stage(x, betas, mlp_params, film_p, fc_w, fc_b):
    B, K, C = x.shape
    nlayers = len(mlp_params)
    cout = mlp_params[-1][0].shape[0]
    nout = fc_w.shape[0]
    Wf, bf, Wh, bh = film_p

    flat_params = []
    specs = []
    for (W, b, g, be, m, v) in mlp_params:
        co = W.shape[0]
        flat_params.append(W)
        specs.append(pl.BlockSpec(W.shape, lambda b_: (0, 0)))
        for p in (b, g, be, m, v):
            flat_params.append(p.reshape(1, co))
            specs.append(pl.BlockSpec((1, co), lambda b_: (0, 0)))
    tail_flat = [Wf, bf.reshape(1, cout), Wh, bh.reshape(1, cout),
                 fc_w, fc_b.reshape(1, nout)]
    tail_specs = [
        pl.BlockSpec(Wf.shape, lambda b_: (0, 0)),
        pl.BlockSpec((1, cout), lambda b_: (0, 0)),
        pl.BlockSpec(Wh.shape, lambda b_: (0, 0)),
        pl.BlockSpec((1, cout), lambda b_: (0, 0)),
        pl.BlockSpec(fc_w.shape, lambda b_: (0, 0)),
        pl.BlockSpec((1, nout), lambda b_: (0, 0)),
    ]

    betas3 = betas.reshape(B, 1, 16)
    out = pl.pallas_call(
        functools.partial(_global_body, nlayers),
        grid=(B,),
        in_specs=[
            pl.BlockSpec((1, K, C), lambda b_: (b_, 0, 0)),
            pl.BlockSpec((1, 1, 16), lambda b_: (b_, 0, 0)),
        ] + specs + tail_specs,
        out_specs=pl.BlockSpec((1, 1, nout), lambda b_: (b_, 0, 0)),
        out_shape=jax.ShapeDtypeStruct((B, 1, nout), jnp.float32),
    )(x, betas3, *flat_params, *tail_flat)
    return out.reshape(B, nout)


def _fps_body(npoint, NBIG, pts_ref, cent_ref, fidx_ref):
    x = pts_ref[0, 0]
    y = pts_ref[0, 1]
    z = pts_ref[0, 2]
    R, C = x.shape
    iot = (lax.broadcasted_iota(jnp.int32, (R, C), 0) * C
           + lax.broadcasted_iota(jnp.int32, (R, C), 1))
    col = lax.broadcasted_iota(jnp.int32, (3, npoint), 1)
    coli = lax.broadcasted_iota(jnp.int32, (1, npoint), 1)

    def body(i, st):
        dist, far, cent, idxs = st
        fm = (iot == far).astype(jnp.float32)
        cx = jnp.sum(x * fm)
        cy = jnp.sum(y * fm)
        cz = jnp.sum(z * fm)
        dxv = x - cx
        dyv = y - cy
        dzv = z - cz
        d = (dxv * dxv + dyv * dyv) + dzv * dzv
        dist = jnp.minimum(dist, d)
        m = jnp.max(dist)
        far_next = jnp.min(jnp.where(dist == m, iot, NBIG))
        cvals = jnp.concatenate(
            [jnp.full((1, 1), cx), jnp.full((1, 1), cy), jnp.full((1, 1), cz)],
            axis=0)
        cent = jnp.where(col == i, cvals, cent)
        idxs = jnp.where(coli == i, far, idxs)
        return (dist, far_next, cent, idxs)

    init = (jnp.full((R, C), 1e10, jnp.float32), jnp.int32(0),
            jnp.zeros((3, npoint), jnp.float32),
            jnp.zeros((1, npoint), jnp.int32))
    _, _, cent, idxs = lax.fori_loop(0, npoint, body, init)
    cent_ref[0] = cent
    fidx_ref[0] = idxs


def _fps_pallas(xyzT, npoint, R, C):
    B = xyzT.shape[0]
    N = R * C
    pts = xyzT.reshape(B, 3, R, C)
    cent, fidx = pl.pallas_call(
        functools.partial(_fps_body, npoint, N),
        grid=(B,),
        in_specs=[pl.BlockSpec((1, 3, R, C), lambda b_: (b_, 0, 0, 0))],
        out_specs=[
            pl.BlockSpec((1, 3, npoint), lambda b_: (b_, 0, 0)),
            pl.BlockSpec((1, 1, npoint), lambda b_: (b_, 0, 0)),
        ],
        out_shape=[
            jax.ShapeDtypeStruct((B, 3, npoint), jnp.float32),
            jax.ShapeDtypeStruct((B, 1, npoint), jnp.int32),
        ],
    )(pts)
    return cent, fidx


def kernel(pointcloud, betas, sa1, sa2, sa3, film_params, fc_w, fc_b):
    xyz = pointcloud

    cent1, _ = _fps_pallas(jnp.transpose(xyz, (0, 2, 1)), 512, 128, 128)
    new_xyz = jnp.transpose(cent1, (0, 2, 1))
    idx = _ball_query(0.2, 64, xyz, new_xyz)
    grouped_xyz = jnp.take_along_axis(xyz[:, None, :, :], idx[..., None], axis=2) - new_xyz[:, :, None, :]
    grouped_abs = jnp.take_along_axis(xyz[:, None, :, :], idx[..., None], axis=2)
    x = jnp.concatenate([grouped_xyz, grouped_abs], axis=-1)
    f1 = _mlp_pool_film(x, betas, sa1, film_params[0], rows_blk=64)

    xyz1 = new_xyz
    cent2, _ = _fps_pallas(jnp.transpose(xyz1, (0, 2, 1)), 256, 8, 64)
    new_xyz2 = jnp.transpose(cent2, (0, 2, 1))
    idx2 = _ball_query(0.4, 64, xyz1, new_xyz2)
    g_xyz2 = jnp.take_along_axis(xyz1[:, None, :, :], idx2[..., None], axis=2) - new_xyz2[:, :, None, :]
    g_f2 = jnp.take_along_axis(f1[:, None, :, :], idx2[..., None], axis=2)
    x2 = jnp.concatenate([g_xyz2, g_f2], axis=-1)
    f2 = _mlp_pool_film(x2, betas, sa2, film_params[1], rows_blk=64)

    x3 = jnp.concatenate([new_xyz2, f2], axis=-1)
    return _global_stage(x3, betas, sa3, film_params[2], fc_w, fc_b)


# speedup vs baseline: 3.6417x; 2.7166x over previous
import functools

import jax
import jax.numpy as jnp
from jax import lax
from jax.experimental import pallas as pl

EPS = 1e-5


def _fps(xyz, npoint):
    B, N, _ = xyz.shape
    def body(i, state):
        idxs, dists, farthest = state
        idxs = idxs.at[:, i].set(farthest)
        centroid = jnp.take_along_axis(xyz, farthest[:, None, None], axis=1)
        d = jnp.sum((xyz - centroid) ** 2, axis=-1)
        dists = jnp.minimum(dists, d)
        farthest = jnp.argmax(dists, axis=-1).astype(jnp.int32)
        return (idxs, dists, farthest)
    init = (jnp.zeros((B, npoint), dtype=jnp.int32),
            jnp.full((B, N), 1e10, dtype=xyz.dtype),
            jnp.zeros((B,), dtype=jnp.int32))
    idxs, _, _ = jax.lax.fori_loop(0, npoint, body, init)
    return idxs


def _ball_query(radius, nsample, xyz, new_xyz):
    N = xyz.shape[1]
    d2 = jnp.sum((new_xyz[:, :, None, :] - xyz[:, None, :, :]) ** 2, axis=-1)
    key = jnp.where(d2 < radius * radius,
                    jnp.arange(N, dtype=jnp.int32)[None, None, :], N)
    negvals, _ = jax.lax.top_k(-key, nsample)
    vals = -negvals
    first = vals[..., :1]
    return jnp.where(vals == N, first, vals).astype(jnp.int32)


def _mlp_body(nlayers, x_ref, betas_ref, *rest):
    params = rest[:nlayers * 6]
    Wf, bf, Wh, bh = rest[nlayers * 6:nlayers * 6 + 4]
    out_ref = rest[-1]

    blk = x_ref.shape[1]
    K = x_ref.shape[2]
    C = x_ref.shape[3]
    h = x_ref[0].reshape(blk * K, C)
    for li in range(nlayers):
        W, b, g, be, m, v = (params[6 * li + j][...] for j in range(6))
        h = jax.lax.dot_general(h, W, (((1,), (1,)), ((), ())),
                                preferred_element_type=jnp.float32)
        h = h + b
        h = (h - m) / jnp.sqrt(v + EPS) * g + be
        h = jax.nn.relu(h)
    cout = h.shape[-1]
    pooled = jnp.max(h.reshape(blk, K, cout), axis=1)
    bvec = betas_ref[0]
    gamma = jax.lax.dot_general(bvec, Wf[...], (((1,), (1,)), ((), ())),
                                preferred_element_type=jnp.float32) + bf[...]
    beta = jax.lax.dot_general(bvec, Wh[...], (((1,), (1,)), ((), ())),
                               preferred_element_type=jnp.float32) + bh[...]
    out_ref[0] = pooled * gamma + beta


def _mlp_pool_film(x, betas, mlp_params, film_p, rows_blk):
    B, S, K, C = x.shape
    nlayers = len(mlp_params)
    cout = mlp_params[-1][0].shape[0]
    Wf, bf, Wh, bh = film_p

    flat_params = []
    specs = []
    for (W, b, g, be, m, v) in mlp_params:
        co = W.shape[0]
        flat_params.append(W)
        specs.append(pl.BlockSpec(W.shape, lambda b_, r_: (0, 0)))
        for p in (b, g, be, m, v):
            flat_params.append(p.reshape(1, co))
            specs.append(pl.BlockSpec((1, co), lambda b_, r_: (0, 0)))
    film_flat = [Wf, bf.reshape(1, cout), Wh, bh.reshape(1, cout)]
    film_specs = [
        pl.BlockSpec(Wf.shape, lambda b_, r_: (0, 0)),
        pl.BlockSpec((1, cout), lambda b_, r_: (0, 0)),
        pl.BlockSpec(Wh.shape, lambda b_, r_: (0, 0)),
        pl.BlockSpec((1, cout), lambda b_, r_: (0, 0)),
    ]

    betas3 = betas.reshape(B, 1, 16)
    grid = (B, S // rows_blk)
    return pl.pallas_call(
        functools.partial(_mlp_body, nlayers),
        grid=grid,
        in_specs=[
            pl.BlockSpec((1, rows_blk, K, C), lambda b_, r_: (b_, r_, 0, 0)),
            pl.BlockSpec((1, 1, 16), lambda b_, r_: (b_, 0, 0)),
        ] + specs + film_specs,
        out_specs=pl.BlockSpec((1, rows_blk, cout), lambda b_, r_: (b_, r_, 0)),
        out_shape=jax.ShapeDtypeStruct((B, S, cout), jnp.float32),
    )(x, betas3, *flat_params, *film_flat)


def _global_body(nlayers, x_ref, betas_ref, *rest):
    params = rest[:nlayers * 6]
    Wf, bf, Wh, bh, fcw, fcb = rest[nlayers * 6:nlayers * 6 + 6]
    out_ref = rest[-1]

    K = x_ref.shape[1]
    C = x_ref.shape[2]
    h = x_ref[0]
    for li in range(nlayers):
        W, b, g, be, m, v = (params[6 * li + j][...] for j in range(6))
        h = jax.lax.dot_general(h, W, (((1,), (1,)), ((), ())),
                                preferred_element_type=jnp.float32)
        h = h + b
        h = (h - m) / jnp.sqrt(v + EPS) * g + be
        h = jax.nn.relu(h)
    pooled = jnp.max(h, axis=0, keepdims=True)
    bvec = betas_ref[0]
    gamma = jax.lax.dot_general(bvec, Wf[...], (((1,), (1,)), ((), ())),
                                preferred_element_type=jnp.float32) + bf[...]
    beta = jax.lax.dot_general(bvec, Wh[...], (((1,), (1,)), ((), ())),
                               preferred_element_type=jnp.float32) + bh[...]
    f = pooled * gamma + beta
    out = jax.lax.dot_general(f, fcw[...], (((1,), (1,)), ((), ())),
                              preferred_element_type=jnp.float32) + fcb[...]
    out_ref[0] = out


def _global_stage(x, betas, mlp_params, film_p, fc_w, fc_b):
    B, K, C = x.shape
    nlayers = len(mlp_params)
    cout = mlp_params[-1][0].shape[0]
    nout = fc_w.shape[0]
    Wf, bf, Wh, bh = film_p

    flat_params = []
    specs = []
    for (W, b, g, be, m, v) in mlp_params:
        co = W.shape[0]
        flat_params.append(W)
        specs.append(pl.BlockSpec(W.shape, lambda b_: (0, 0)))
        for p in (b, g, be, m, v):
            flat_params.append(p.reshape(1, co))
            specs.append(pl.BlockSpec((1, co), lambda b_: (0, 0)))
    tail_flat = [Wf, bf.reshape(1, cout), Wh, bh.reshape(1, cout),
                 fc_w, fc_b.reshape(1, nout)]
    tail_specs = [
        pl.BlockSpec(Wf.shape, lambda b_: (0, 0)),
        pl.BlockSpec((1, cout), lambda b_: (0, 0)),
        pl.BlockSpec(Wh.shape, lambda b_: (0, 0)),
        pl.BlockSpec((1, cout), lambda b_: (0, 0)),
        pl.BlockSpec(fc_w.shape, lambda b_: (0, 0)),
        pl.BlockSpec((1, nout), lambda b_: (0, 0)),
    ]

    betas3 = betas.reshape(B, 1, 16)
    out = pl.pallas_call(
        functools.partial(_global_body, nlayers),
        grid=(B,),
        in_specs=[
            pl.BlockSpec((1, K, C), lambda b_: (b_, 0, 0)),
            pl.BlockSpec((1, 1, 16), lambda b_: (b_, 0, 0)),
        ] + specs + tail_specs,
        out_specs=pl.BlockSpec((1, 1, nout), lambda b_: (b_, 0, 0)),
        out_shape=jax.ShapeDtypeStruct((B, 1, nout), jnp.float32),
    )(x, betas3, *flat_params, *tail_flat)
    return out.reshape(B, nout)


def _fps_body(npoint, NBIG, pts_ref, cent_ref, fidx_ref):
    x = pts_ref[0, 0]
    y = pts_ref[0, 1]
    z = pts_ref[0, 2]
    R, C = x.shape
    iot = (lax.broadcasted_iota(jnp.int32, (R, C), 0) * C
           + lax.broadcasted_iota(jnp.int32, (R, C), 1))
    col = lax.broadcasted_iota(jnp.int32, (3, npoint), 1)
    coli = lax.broadcasted_iota(jnp.int32, (1, npoint), 1)

    def body(i, st):
        dist, far, cent, idxs = st
        fm = (iot == far).astype(jnp.float32)
        cx = jnp.sum(x * fm)
        cy = jnp.sum(y * fm)
        cz = jnp.sum(z * fm)
        dxv = x - cx
        dyv = y - cy
        dzv = z - cz
        d = (dxv * dxv + dyv * dyv) + dzv * dzv
        dist = jnp.minimum(dist, d)
        m = jnp.max(dist)
        far_next = jnp.min(jnp.where(dist == m, iot, NBIG))
        cvals = jnp.concatenate(
            [jnp.full((1, 1), cx), jnp.full((1, 1), cy), jnp.full((1, 1), cz)],
            axis=0)
        cent = jnp.where(col == i, cvals, cent)
        idxs = jnp.where(coli == i, far, idxs)
        return (dist, far_next, cent, idxs)

    init = (jnp.full((R, C), 1e10, jnp.float32), jnp.int32(0),
            jnp.zeros((3, npoint), jnp.float32),
            jnp.zeros((1, npoint), jnp.int32))
    _, _, cent, idxs = lax.fori_loop(0, npoint, body, init)
    cent_ref[0] = cent
    fidx_ref[0] = idxs


def _fps_pallas(xyzT, npoint, R, C):
    B = xyzT.shape[0]
    N = R * C
    pts = xyzT.reshape(B, 3, R, C)
    cent, fidx = pl.pallas_call(
        functools.partial(_fps_body, npoint, N),
        grid=(B,),
        in_specs=[pl.BlockSpec((1, 3, R, C), lambda b_: (b_, 0, 0, 0))],
        out_specs=[
            pl.BlockSpec((1, 3, npoint), lambda b_: (b_, 0, 0)),
            pl.BlockSpec((1, 1, npoint), lambda b_: (b_, 0, 0)),
        ],
        out_shape=[
            jax.ShapeDtypeStruct((B, 3, npoint), jnp.float32),
            jax.ShapeDtypeStruct((B, 1, npoint), jnp.int32),
        ],
    )(pts)
    return cent, fidx


def _bq_body(R2, K, pts_ref, cent_ref, idx_ref):
    x = pts_ref[0, 0]
    y = pts_ref[0, 1]
    z = pts_ref[0, 2]
    N = x.shape[0] * x.shape[1]
    rows = cent_ref.shape[2]
    G = N // 128
    xf = x.reshape(1, N)
    yf = y.reshape(1, N)
    zf = z.reshape(1, N)
    cx = cent_ref[0, 0].reshape(rows, 1)
    cy = cent_ref[0, 1].reshape(rows, 1)
    cz = cent_ref[0, 2].reshape(rows, 1)
    dx = cx - xf
    dy = cy - yf
    dz = cz - zf
    d2 = (dx * dx + dy * dy) + dz * dz
    mi = (d2 < R2).astype(jnp.float32)
    mi3 = mi.reshape(rows, G, 128)
    li = lax.broadcasted_iota(jnp.int32, (128, 128), 0)
    mj = lax.broadcasted_iota(jnp.int32, (128, 128), 1)
    Tinc = (li <= mj).astype(jnp.float32)
    within = lax.dot_general(mi3, Tinc, (((2,), (0,)), ((), ())),
                             preferred_element_type=jnp.float32)
    totals = within[:, :, 127]
    gi = lax.broadcasted_iota(jnp.int32, (G, G), 0)
    gj = lax.broadcasted_iota(jnp.int32, (G, G), 1)
    SL = (gi < gj).astype(jnp.float32)
    base = lax.dot_general(totals, SL, (((1,), (0,)), ((), ())),
                           preferred_element_type=jnp.float32)
    cumi = base + totals
    count = base[:, G - 1:G] + totals[:, G - 1:G]
    kv = lax.broadcasted_iota(jnp.int32, (rows, K), 1).astype(jnp.float32)
    gk = jnp.sum((cumi[:, None, :] <= kv[:, :, None]).astype(jnp.float32),
                 axis=2).astype(jnp.int32)
    bk = jnp.take_along_axis(base, gk, axis=1)
    kp = kv - bk
    giota = lax.broadcasted_iota(jnp.int32, (rows, K, G), 2)
    onehot = (giota == gk[:, :, None]).astype(jnp.float32)
    wrows = lax.dot_general(onehot, within,
                            (((2,), (1,)), ((0,), (0,))),
                            preferred_element_type=jnp.float32)
    pos = jnp.sum((wrows <= kp[:, :, None]).astype(jnp.float32), axis=2)
    idx = gk * 128 + pos.astype(jnp.int32)
    first = idx[:, 0:1]
    idx = jnp.where(kv < count, idx, first)
    idx_ref[0] = idx


def _bq_pallas(xyzT, cent, R2, K, R, C):
    B, _, N = xyzT.shape
    S = cent.shape[2]
    rows = 128 if S % 128 == 0 else S
    pts = xyzT.reshape(B, 3, R, C)
    idx = pl.pallas_call(
        functools.partial(_bq_body, R2, K),
        grid=(B, S // rows),
        in_specs=[
            pl.BlockSpec((1, 3, R, C), lambda b_, r_: (b_, 0, 0, 0)),
            pl.BlockSpec((1, 3, rows), lambda b_, r_: (b_, 0, r_)),
        ],
        out_specs=pl.BlockSpec((1, rows, K), lambda b_, r_: (b_, r_, 0)),
        out_shape=jax.ShapeDtypeStruct((B, S, K), jnp.int32),
    )(pts, cent)
    return idx


def kernel(pointcloud, betas, sa1, sa2, sa3, film_params, fc_w, fc_b):
    xyz = pointcloud

    cent1, _ = _fps_pallas(jnp.transpose(xyz, (0, 2, 1)), 512, 128, 128)
    new_xyz = jnp.transpose(cent1, (0, 2, 1))
    idx = _bq_pallas(jnp.transpose(xyz, (0, 2, 1)), cent1, 0.2 * 0.2, 64, 128, 128)
    grouped_xyz = jnp.take_along_axis(xyz[:, None, :, :], idx[..., None], axis=2) - new_xyz[:, :, None, :]
    grouped_abs = jnp.take_along_axis(xyz[:, None, :, :], idx[..., None], axis=2)
    x = jnp.concatenate([grouped_xyz, grouped_abs], axis=-1)
    f1 = _mlp_pool_film(x, betas, sa1, film_params[0], rows_blk=64)

    xyz1 = new_xyz
    cent2, _ = _fps_pallas(jnp.transpose(xyz1, (0, 2, 1)), 256, 8, 64)
    new_xyz2 = jnp.transpose(cent2, (0, 2, 1))
    idx2 = _bq_pallas(jnp.transpose(xyz1, (0, 2, 1)), cent2, 0.4 * 0.4, 64, 8, 64)
    g_xyz2 = jnp.take_along_axis(xyz1[:, None, :, :], idx2[..., None], axis=2) - new_xyz2[:, :, None, :]
    g_f2 = jnp.take_along_axis(f1[:, None, :, :], idx2[..., None], axis=2)
    x2 = jnp.concatenate([g_xyz2, g_f2], axis=-1)
    f2 = _mlp_pool_film(x2, betas, sa2, film_params[1], rows_blk=64)

    x3 = jnp.concatenate([new_xyz2, f2], axis=-1)
    return _global_stage(x3, betas, sa3, film_params[2], fc_w, fc_b)


# speedup vs baseline: 4.5736x; 1.2559x over previous
import functools

import jax
import jax.numpy as jnp
from jax import lax
from jax.experimental import pallas as pl

EPS = 1e-5


def _fps(xyz, npoint):
    B, N, _ = xyz.shape
    def body(i, state):
        idxs, dists, farthest = state
        idxs = idxs.at[:, i].set(farthest)
        centroid = jnp.take_along_axis(xyz, farthest[:, None, None], axis=1)
        d = jnp.sum((xyz - centroid) ** 2, axis=-1)
        dists = jnp.minimum(dists, d)
        farthest = jnp.argmax(dists, axis=-1).astype(jnp.int32)
        return (idxs, dists, farthest)
    init = (jnp.zeros((B, npoint), dtype=jnp.int32),
            jnp.full((B, N), 1e10, dtype=xyz.dtype),
            jnp.zeros((B,), dtype=jnp.int32))
    idxs, _, _ = jax.lax.fori_loop(0, npoint, body, init)
    return idxs


def _ball_query(radius, nsample, xyz, new_xyz):
    N = xyz.shape[1]
    d2 = jnp.sum((new_xyz[:, :, None, :] - xyz[:, None, :, :]) ** 2, axis=-1)
    key = jnp.where(d2 < radius * radius,
                    jnp.arange(N, dtype=jnp.int32)[None, None, :], N)
    negvals, _ = jax.lax.top_k(-key, nsample)
    vals = -negvals
    first = vals[..., :1]
    return jnp.where(vals == N, first, vals).astype(jnp.int32)


def _mlp_body(nlayers, x_ref, betas_ref, *rest):
    params = rest[:nlayers * 6]
    Wf, bf, Wh, bh = rest[nlayers * 6:nlayers * 6 + 4]
    out_ref = rest[-1]

    blk = x_ref.shape[1]
    K = x_ref.shape[2]
    C = x_ref.shape[3]
    h = x_ref[0].reshape(blk * K, C)
    for li in range(nlayers):
        W, b, g, be, m, v = (params[6 * li + j][...] for j in range(6))
        h = jax.lax.dot_general(h, W, (((1,), (1,)), ((), ())),
                                preferred_element_type=jnp.float32)
        h = h + b
        h = (h - m) / jnp.sqrt(v + EPS) * g + be
        h = jax.nn.relu(h)
    cout = h.shape[-1]
    pooled = jnp.max(h.reshape(blk, K, cout), axis=1)
    bvec = betas_ref[0]
    gamma = jax.lax.dot_general(bvec, Wf[...], (((1,), (1,)), ((), ())),
                                preferred_element_type=jnp.float32) + bf[...]
    beta = jax.lax.dot_general(bvec, Wh[...], (((1,), (1,)), ((), ())),
                               preferred_element_type=jnp.float32) + bh[...]
    out_ref[0] = pooled * gamma + beta


def _mlp_pool_film(x, betas, mlp_params, film_p, rows_blk):
    B, S, K, C = x.shape
    nlayers = len(mlp_params)
    cout = mlp_params[-1][0].shape[0]
    Wf, bf, Wh, bh = film_p

    flat_params = []
    specs = []
    for (W, b, g, be, m, v) in mlp_params:
        co = W.shape[0]
        flat_params.append(W)
        specs.append(pl.BlockSpec(W.shape, lambda b_, r_: (0, 0)))
        for p in (b, g, be, m, v):
            flat_params.append(p.reshape(1, co))
            specs.append(pl.BlockSpec((1, co), lambda b_, r_: (0, 0)))
    film_flat = [Wf, bf.reshape(1, cout), Wh, bh.reshape(1, cout)]
    film_specs = [
        pl.BlockSpec(Wf.shape, lambda b_, r_: (0, 0)),
        pl.BlockSpec((1, cout), lambda b_, r_: (0, 0)),
        pl.BlockSpec(Wh.shape, lambda b_, r_: (0, 0)),
        pl.BlockSpec((1, cout), lambda b_, r_: (0, 0)),
    ]

    betas3 = betas.reshape(B, 1, 16)
    grid = (B, S // rows_blk)
    return pl.pallas_call(
        functools.partial(_mlp_body, nlayers),
        grid=grid,
        in_specs=[
            pl.BlockSpec((1, rows_blk, K, C), lambda b_, r_: (b_, r_, 0, 0)),
            pl.BlockSpec((1, 1, 16), lambda b_, r_: (b_, 0, 0)),
        ] + specs + film_specs,
        out_specs=pl.BlockSpec((1, rows_blk, cout), lambda b_, r_: (b_, r_, 0)),
        out_shape=jax.ShapeDtypeStruct((B, S, cout), jnp.float32),
    )(x, betas3, *flat_params, *film_flat)


def _global_body(nlayers, x_ref, betas_ref, *rest):
    params = rest[:nlayers * 6]
    Wf, bf, Wh, bh, fcw, fcb = rest[nlayers * 6:nlayers * 6 + 6]
    out_ref = rest[-1]

    K = x_ref.shape[1]
    C = x_ref.shape[2]
    h = x_ref[0]
    for li in range(nlayers):
        W, b, g, be, m, v = (params[6 * li + j][...] for j in range(6))
        h = jax.lax.dot_general(h, W, (((1,), (1,)), ((), ())),
                                preferred_element_type=jnp.float32)
        h = h + b
        h = (h - m) / jnp.sqrt(v + EPS) * g + be
        h = jax.nn.relu(h)
    pooled = jnp.max(h, axis=0, keepdims=True)
    bvec = betas_ref[0]
    gamma = jax.lax.dot_general(bvec, Wf[...], (((1,), (1,)), ((), ())),
                                preferred_element_type=jnp.float32) + bf[...]
    beta = jax.lax.dot_general(bvec, Wh[...], (((1,), (1,)), ((), ())),
                               preferred_element_type=jnp.float32) + bh[...]
    f = pooled * gamma + beta
    out = jax.lax.dot_general(f, fcw[...], (((1,), (1,)), ((), ())),
                              preferred_element_type=jnp.float32) + fcb[...]
    out_ref[0] = out


def _global_stage(x, betas, mlp_params, film_p, fc_w, fc_b):
    B, K, C = x.shape
    nlayers = len(mlp_params)
    cout = mlp_params[-1][0].shape[0]
    nout = fc_w.shape[0]
    Wf, bf, Wh, bh = film_p

    flat_params = []
    specs = []
    for (W, b, g, be, m, v) in mlp_params:
        co = W.shape[0]
        flat_params.append(W)
        specs.append(pl.BlockSpec(W.shape, lambda b_: (0, 0)))
        for p in (b, g, be, m, v):
            flat_params.append(p.reshape(1, co))
            specs.append(pl.BlockSpec((1, co), lambda b_: (0, 0)))
    tail_flat = [Wf, bf.reshape(1, cout), Wh, bh.reshape(1, cout),
                 fc_w, fc_b.reshape(1, nout)]
    tail_specs = [
        pl.BlockSpec(Wf.shape, lambda b_: (0, 0)),
        pl.BlockSpec((1, cout), lambda b_: (0, 0)),
        pl.BlockSpec(Wh.shape, lambda b_: (0, 0)),
        pl.BlockSpec((1, cout), lambda b_: (0, 0)),
        pl.BlockSpec(fc_w.shape, lambda b_: (0, 0)),
        pl.BlockSpec((1, nout), lambda b_: (0, 0)),
    ]

    betas3 = betas.reshape(B, 1, 16)
    out = pl.pallas_call(
        functools.partial(_global_body, nlayers),
        grid=(B,),
        in_specs=[
            pl.BlockSpec((1, K, C), lambda b_: (b_, 0, 0)),
            pl.BlockSpec((1, 1, 16), lambda b_: (b_, 0, 0)),
        ] + specs + tail_specs,
        out_specs=pl.BlockSpec((1, 1, nout), lambda b_: (b_, 0, 0)),
        out_shape=jax.ShapeDtypeStruct((B, 1, nout), jnp.float32),
    )(x, betas3, *flat_params, *tail_flat)
    return out.reshape(B, nout)


def _fpsb_body(npoint, NBIG, pts_ref, cent_ref):
    x = pts_ref[:, 0]
    y = pts_ref[:, 1]
    z = pts_ref[:, 2]
    B, R, C = x.shape
    iot = (lax.broadcasted_iota(jnp.int32, (B, R, C), 1) * C
           + lax.broadcasted_iota(jnp.int32, (B, R, C), 2))
    col = lax.broadcasted_iota(jnp.int32, (B, 3, npoint), 2)

    def body(i, st):
        dist, far, cent = st
        fm = (iot == far).astype(jnp.float32)
        cx = jnp.sum(x * fm, axis=(1, 2), keepdims=True)
        cy = jnp.sum(y * fm, axis=(1, 2), keepdims=True)
        cz = jnp.sum(z * fm, axis=(1, 2), keepdims=True)
        dxv = x - cx
        dyv = y - cy
        dzv = z - cz
        d = (dxv * dxv + dyv * dyv) + dzv * dzv
        dist = jnp.minimum(dist, d)
        m = jnp.max(dist, axis=(1, 2), keepdims=True)
        far_next = jnp.min(jnp.where(dist == m, iot, NBIG),
                           axis=(1, 2), keepdims=True)
        cvals = jnp.concatenate([cx, cy, cz], axis=1)
        cent = jnp.where(col == i, cvals, cent)
        return (dist, far_next, cent)

    init = (jnp.full((B, R, C), 1e10, jnp.float32),
            jnp.zeros((B, 1, 1), jnp.int32),
            jnp.zeros((B, 3, npoint), jnp.float32))
    _, _, cent = lax.fori_loop(0, npoint, body, init)
    cent_ref[...] = cent


def _fps_pallas(xyzT, npoint, R, C):
    B = xyzT.shape[0]
    N = R * C
    pts = xyzT.reshape(B, 3, R, C)
    cent = pl.pallas_call(
        functools.partial(_fpsb_body, npoint, N),
        grid=(1,),
        in_specs=[pl.BlockSpec((B, 3, R, C), lambda _: (0, 0, 0, 0))],
        out_specs=pl.BlockSpec((B, 3, npoint), lambda _: (0, 0, 0)),
        out_shape=jax.ShapeDtypeStruct((B, 3, npoint), jnp.float32),
    )(pts)
    return cent, None


def _bq_body(R2, K, pts_ref, cent_ref, idx_ref):
    x = pts_ref[0, 0]
    y = pts_ref[0, 1]
    z = pts_ref[0, 2]
    N = x.shape[0] * x.shape[1]
    rows = cent_ref.shape[2]
    G = N // 128
    xf = x.reshape(1, N)
    yf = y.reshape(1, N)
    zf = z.reshape(1, N)
    cx = cent_ref[0, 0].reshape(rows, 1)
    cy = cent_ref[0, 1].reshape(rows, 1)
    cz = cent_ref[0, 2].reshape(rows, 1)
    dx = cx - xf
    dy = cy - yf
    dz = cz - zf
    d2 = (dx * dx + dy * dy) + dz * dz
    mi = (d2 < R2).astype(jnp.float32)
    mi3 = mi.reshape(rows, G, 128)
    li = lax.broadcasted_iota(jnp.int32, (128, 128), 0)
    mj = lax.broadcasted_iota(jnp.int32, (128, 128), 1)
    Tinc = (li <= mj).astype(jnp.float32)
    within = lax.dot_general(mi3, Tinc, (((2,), (0,)), ((), ())),
                             preferred_element_type=jnp.float32)
    totals = within[:, :, 127]
    gi = lax.broadcasted_iota(jnp.int32, (G, G), 0)
    gj = lax.broadcasted_iota(jnp.int32, (G, G), 1)
    SL = (gi < gj).astype(jnp.float32)
    base = lax.dot_general(totals, SL, (((1,), (0,)), ((), ())),
                           preferred_element_type=jnp.float32)
    cumi = base + totals
    count = base[:, G - 1:G] + totals[:, G - 1:G]
    kv = lax.broadcasted_iota(jnp.int32, (rows, K), 1).astype(jnp.float32)
    gk = jnp.sum((cumi[:, None, :] <= kv[:, :, None]).astype(jnp.float32),
                 axis=2).astype(jnp.int32)
    bk = jnp.take_along_axis(base, gk, axis=1)
    kp = kv - bk
    giota = lax.broadcasted_iota(jnp.int32, (rows, K, G), 2)
    onehot = (giota == gk[:, :, None]).astype(jnp.float32)
    wrows = lax.dot_general(onehot, within,
                            (((2,), (1,)), ((0,), (0,))),
                            preferred_element_type=jnp.float32)
    pos = jnp.sum((wrows <= kp[:, :, None]).astype(jnp.float32), axis=2)
    idx = gk * 128 + pos.astype(jnp.int32)
    first = idx[:, 0:1]
    idx = jnp.where(kv < count, idx, first)
    idx_ref[0] = idx


def _bq_pallas(xyzT, cent, R2, K, R, C):
    B, _, N = xyzT.shape
    S = cent.shape[2]
    rows = 128 if S % 128 == 0 else S
    pts = xyzT.reshape(B, 3, R, C)
    idx = pl.pallas_call(
        functools.partial(_bq_body, R2, K),
        grid=(B, S // rows),
        in_specs=[
            pl.BlockSpec((1, 3, R, C), lambda b_, r_: (b_, 0, 0, 0)),
            pl.BlockSpec((1, 3, rows), lambda b_, r_: (b_, 0, r_)),
        ],
        out_specs=pl.BlockSpec((1, rows, K), lambda b_, r_: (b_, r_, 0)),
        out_shape=jax.ShapeDtypeStruct((B, S, K), jnp.int32),
    )(pts, cent)
    return idx


def kernel(pointcloud, betas, sa1, sa2, sa3, film_params, fc_w, fc_b):
    xyz = pointcloud

    cent1, _ = _fps_pallas(jnp.transpose(xyz, (0, 2, 1)), 512, 128, 128)
    new_xyz = jnp.transpose(cent1, (0, 2, 1))
    idx = _bq_pallas(jnp.transpose(xyz, (0, 2, 1)), cent1, 0.2 * 0.2, 64, 128, 128)
    grouped_xyz = jnp.take_along_axis(xyz[:, None, :, :], idx[..., None], axis=2) - new_xyz[:, :, None, :]
    grouped_abs = jnp.take_along_axis(xyz[:, None, :, :], idx[..., None], axis=2)
    x = jnp.concatenate([grouped_xyz, grouped_abs], axis=-1)
    f1 = _mlp_pool_film(x, betas, sa1, film_params[0], rows_blk=64)

    xyz1 = new_xyz
    cent2, _ = _fps_pallas(jnp.transpose(xyz1, (0, 2, 1)), 256, 8, 64)
    new_xyz2 = jnp.transpose(cent2, (0, 2, 1))
    idx2 = _bq_pallas(jnp.transpose(xyz1, (0, 2, 1)), cent2, 0.4 * 0.4, 64, 8, 64)
    g_xyz2 = jnp.take_along_axis(xyz1[:, None, :, :], idx2[..., None], axis=2) - new_xyz2[:, :, None, :]
    g_f2 = jnp.take_along_axis(f1[:, None, :, :], idx2[..., None], axis=2)
    x2 = jnp.concatenate([g_xyz2, g_f2], axis=-1)
    f2 = _mlp_pool_film(x2, betas, sa2, film_params[1], rows_blk=64)

    x3 = jnp.concatenate([new_xyz2, f2], axis=-1)
    return _global_stage(x3, betas, sa3, film_params[2], fc_w, fc_b)
